# baseline (speedup 1.0000x reference)
import functools

import jax
import jax.numpy as jnp
from jax import lax
from jax.experimental import pallas as pl
from jax.experimental.pallas import tpu as pltpu
from jax.experimental.pallas import tpu_sc as plsc

_EMB = 32
_B = 16384
_NROWS = 1000000
_PACK = 4
_PROWS = _NROWS // _PACK
_NUMF = 16
_HID = 128
_NCLS = 2
_NC, _NS = 2, 16
_NW = _NC * _NS
_BPW = _B // _NW
_CH = 128
_NCH = _BPW // _CH
_LANES = 16


_TBLK = 7813
_TMAIN = _TBLK - 1
_KC = 4
_NGRP = _TMAIN // _KC
_SUP = 21


def _shuffle_body(srcT, tgtT, stail, ttail, s4, t4, tin, outb, sem, osem):
    wid = lax.axis_index("s") * _NC + lax.axis_index("c")

    iot = jax.lax.broadcasted_iota(jnp.int32, (_LANES,), 0)

    def lane_consts(x):
        L = iot + _LANES * x
        q = L >> 5
        rem = L & 31
        return rem >> 3, rem & 7, q

    consts = [lane_consts(x) for x in range(8)]

    def fire_in(tab, t, slot):
        g = t * _NW + wid

        @pl.when(g < _NGRP)
        def _():
            lane0 = pl.multiple_of(g * (128 * _KC), 128)
            for b in range(4):
                pltpu.async_copy(
                    tab.at[pl.ds(8 * b, 8), pl.ds(lane0, 128 * _KC)],
                    tin.at[slot, b], sem)

    def wait_in(tab, t, slot):
        g = t * _NW + wid

        @pl.when(g < _NGRP)
        def _():
            for b in range(4):
                pltpu.make_async_copy(
                    tab.at[pl.ds(0, 8), pl.ds(0, 128 * _KC)],
                    tin.at[slot, b], sem).wait()

    def wait_out(tab, out, t, slot):
        g = t * _NW + wid

        @pl.when(jnp.logical_and(g >= 0, g < _NGRP))
        def _():
            pltpu.make_async_copy(
                out.at[pl.ds(0, _EMB * _KC)], outb.at[slot], osem).wait()

    for tab, out in ((srcT, s4), (tgtT, t4)):
        def step(tab, out, t, slot):
            fire_in(tab, t + 2, (slot + 2) % 3)
            wait_in(tab, t, slot)
            wait_out(tab, out, t - 3, slot)
            g = t * _NW + wid

            @pl.when(g < _NGRP)
            def _():
                def prow(p2, cr):
                    coff = 128 * (p2 >> 5) + 4 * (p2 & 31)
                    for x in range(8):
                        b0, r0, q0 = consts[x]
                        v = plsc.load_gather(tin.at[slot],
                                             [b0, r0, q0 + coff])
                        outb[slot, p2, pl.ds(_LANES * x, _LANES)] = v
                    return cr

                lax.fori_loop(0, _EMB * _KC, prow, 0)
                row0 = pl.multiple_of(g * (_EMB * _KC), 8)
                pltpu.async_copy(outb.at[slot],
                                 out.at[pl.ds(row0, _EMB * _KC)], osem)

        def sup(ts, carry, tab=tab, out=out):
            for phase in range(3):
                step(tab, out, 3 * ts + phase, phase)
            return carry

        fire_in(tab, 0, 0)
        fire_in(tab, 1, 1)
        lax.fori_loop(0, _SUP, sup, 0)
        for last in range(3):
            wait_out(tab, out, 3 * _SUP - 3 + last, last)

    @pl.when(wid == 0)
    def _tail():
        pltpu.sync_copy(stail, outb.at[0, pl.ds(0, 16)])
        pltpu.sync_copy(outb.at[0, pl.ds(0, 16)],
                        s4.at[pl.ds(_TMAIN * _EMB, 16)])
        pltpu.sync_copy(ttail, outb.at[0, pl.ds(0, 16)])
        pltpu.sync_copy(outb.at[0, pl.ds(0, 16)],
                        t4.at[pl.ds(_TMAIN * _EMB, 16)])


_shuffle_cache = []


def _shuffle(*args):
    if not _shuffle_cache:
        mesh = plsc.VectorSubcoreMesh(
            core_axis_name="c", subcore_axis_name="s",
            num_cores=_NC, num_subcores=_NS,
        )
        _shuffle_cache.append(pl.kernel(
            _shuffle_body,
            out_type=(
                jax.ShapeDtypeStruct((_PROWS, 128), jnp.float32),
                jax.ShapeDtypeStruct((_PROWS, 128), jnp.float32),
            ),
            mesh=mesh,
            scratch_types=[
                pltpu.VMEM((3, 4, 8, 128 * _KC), jnp.float32),
                pltpu.VMEM((3, _EMB * _KC, 128), jnp.float32),
                pltpu.SemaphoreType.DMA,
                pltpu.SemaphoreType.DMA,
            ],
            compiler_params=pltpu.CompilerParams(needs_layout_passes=False),
        ))
    return _shuffle_cache[0](*args)


def _gather_body(sidx_hbm, tidx_hbm, src4, tgt4, souT, touT,
                 sidx_v, tidx_v, srow, trow, sph, tph,
                 sbufs, tbufs, soutT, toutT, sem):
    wid = lax.axis_index("s") * _NC + lax.axis_index("c")
    pltpu.sync_copy(sidx_hbm.at[wid], sidx_v)
    pltpu.sync_copy(tidx_hbm.at[wid], tidx_v)

    for idx_v, row_v, ph_v in ((sidx_v, srow, sph), (tidx_v, trow, tph)):
        for k in range(_BPW // _LANES):
            sl = pl.ds(k * _LANES, _LANES)
            i = idx_v[sl]
            row_v[sl] = i >> 2
            ph_v[sl] = i & 3

    def fire(j):
        isl = pl.ds(j * _CH, _CH)
        return (pltpu.async_copy(src4.at[srow.at[isl]], sbufs.at[j % 2], sem),
                pltpu.async_copy(tgt4.at[trow.at[isl]], tbufs.at[j % 2], sem))

    inflight = {0: fire(0), 1: fire(1)}
    for j in range(_NCH):
        cs, ct = inflight.pop(j)
        cs.wait()
        ct.wait()
        for buf_pair, ph_v, outT in ((sbufs, sph, soutT),
                                     (tbufs, tph, toutT)):
            rows_v = buf_pair.at[j % 2]

            def extract(k, carry, rows_v=rows_v, ph_v=ph_v, outT=outT, j=j):
                rid = jax.lax.broadcasted_iota(jnp.int32, (_LANES,), 0) \
                    + k * _LANES
                ph = plsc.load_gather(ph_v, [j * _CH + rid])
                lane0 = ph * _EMB
                for d in range(_EMB):
                    vals = plsc.load_gather(rows_v, [rid, lane0 + d])
                    outT[d, pl.ds(j * _CH + k * _LANES, _LANES)] = vals
                return carry

            lax.fori_loop(0, _CH // _LANES, extract, 0)
        if j + 2 < _NCH:
            inflight[j + 2] = fire(j + 2)

    pltpu.sync_copy(soutT, souT.at[:, pl.ds(wid * _BPW, _BPW)])
    pltpu.sync_copy(toutT, touT.at[:, pl.ds(wid * _BPW, _BPW)])


_gather_cache = []


def _gather(*args):
    if not _gather_cache:
        mesh = plsc.VectorSubcoreMesh(
            core_axis_name="c", subcore_axis_name="s",
            num_cores=_NC, num_subcores=_NS,
        )
        _gather_cache.append(pl.kernel(
            _gather_body,
            out_type=(
                jax.ShapeDtypeStruct((_EMB, _B), jnp.float32),
                jax.ShapeDtypeStruct((_EMB, _B), jnp.float32),
            ),
            mesh=mesh,
            scratch_types=[
                pltpu.VMEM((_BPW,), jnp.int32),
                pltpu.VMEM((_BPW,), jnp.int32),
                pltpu.VMEM((_BPW,), jnp.int32),
                pltpu.VMEM((_BPW,), jnp.int32),
                pltpu.VMEM((_BPW,), jnp.int32),
                pltpu.VMEM((_BPW,), jnp.int32),
                pltpu.VMEM((2, _CH, 128), jnp.float32),
                pltpu.VMEM((2, _CH, 128), jnp.float32),
                pltpu.VMEM((_EMB, _BPW), jnp.float32),
                pltpu.VMEM((_EMB, _BPW), jnp.float32),
                pltpu.SemaphoreType.DMA,
            ],
            compiler_params=pltpu.CompilerParams(
                use_tc_tiling_on_sc=False, needs_layout_passes=False),
        ))
    return _gather_cache[0](*args)


def _mlp_body(sT, tT, n, w1s, w1t, w1n, b1, w2, b2, o):
    cdim = (((0,), (0,)), ((), ()))
    h = (lax.dot_general(sT[...], w1s[...], cdim,
                         preferred_element_type=jnp.float32)
         + lax.dot_general(tT[...], w1t[...], cdim,
                           preferred_element_type=jnp.float32)
         + jnp.dot(n[...], w1n[...], preferred_element_type=jnp.float32)
         + b1[...])
    h = jnp.maximum(h, 0.0)
    o[...] = jnp.dot(h, w2[...], preferred_element_type=jnp.float32) + b2[...]


_BLK = 2048


def _mlp(sT, tT, n, w1s, w1t, w1n, b1, w2, b2):
    grid = (_B // _BLK,)
    full = lambda i: (0, 0)
    return pl.pallas_call(
        _mlp_body,
        grid=grid,
        in_specs=[
            pl.BlockSpec((_EMB, _BLK), lambda i: (0, i)),
            pl.BlockSpec((_EMB, _BLK), lambda i: (0, i)),
            pl.BlockSpec((_BLK, _NUMF), lambda i: (i, 0)),
            pl.BlockSpec((_EMB, _HID), full),
            pl.BlockSpec((_EMB, _HID), full),
            pl.BlockSpec((_NUMF, _HID), full),
            pl.BlockSpec((1, _HID), full),
            pl.BlockSpec((_HID, _NCLS), full),
            pl.BlockSpec((1, _NCLS), full),
        ],
        out_specs=pl.BlockSpec((_BLK, _NCLS), lambda i: (i, 0)),
        out_shape=jax.ShapeDtypeStruct((_B, _NCLS), jnp.float32),
    )(sT, tT, n, w1s, w1t, w1n, b1, w2, b2)


def kernel(cat_feats, num_feats, src_emb, tgt_emb, W1, b1, W2, b2):
    src_id = cat_feats[:, 0].reshape(_NW, _BPW)
    tgt_id = cat_feats[:, 1].reshape(_NW, _BPW)
    stail = src_emb[_TMAIN * 128:].reshape(16, 128)
    ttail = tgt_emb[_TMAIN * 128:].reshape(16, 128)
    s4, t4 = _shuffle(src_emb.T, tgt_emb.T, stail, ttail)
    sT, tT = _gather(src_id, tgt_id, s4, t4)
    w1s = W1[:, :_EMB].T
    w1t = W1[:, _EMB:2 * _EMB].T
    w1n = W1[:, 2 * _EMB:].T
    return _mlp(sT, tT, num_feats, w1s, w1t, w1n,
                b1.reshape(1, _HID), W2.T, b2.reshape(1, _NCLS))


# speedup vs baseline: 1.7380x; 1.7380x over previous
import functools

import jax
import jax.numpy as jnp
from jax import lax
from jax.experimental import pallas as pl
from jax.experimental.pallas import tpu as pltpu
from jax.experimental.pallas import tpu_sc as plsc

_EMB = 32
_B = 16384
_NUMF = 16
_HID = 128
_NCLS = 2
_NC, _NS = 2, 16
_NW = _NC * _NS
_BPW = _B // _NW
_CH = 128
_NCH = _BPW // _CH

def _gather_body(src_id, tgt_id, src_emb, tgt_emb, src_out, tgt_out,
                 sidx, tidx, srows, trows, sem):
    wid = lax.axis_index("s") * _NC + lax.axis_index("c")
    pltpu.sync_copy(src_id.at[wid], sidx)
    pltpu.sync_copy(tgt_id.at[wid], tidx)
    copies = []
    for j in range(_NCH):
        copies.append(pltpu.async_copy(src_emb.at[sidx.at[j]], srows.at[j], sem))
        copies.append(pltpu.async_copy(tgt_emb.at[tidx.at[j]], trows.at[j], sem))
    for c in copies:
        c.wait()
    pltpu.sync_copy(srows, src_out.at[wid])
    pltpu.sync_copy(trows, tgt_out.at[wid])


_gather_cache = []


def _gather(*args):
    if not _gather_cache:
        mesh = plsc.VectorSubcoreMesh(
            core_axis_name="c", subcore_axis_name="s",
            num_cores=_NC, num_subcores=_NS,
        )
        _gather_cache.append(pl.kernel(
            _gather_body,
            out_type=(
                jax.ShapeDtypeStruct((_NW, _NCH, _CH, _EMB), jnp.float32),
                jax.ShapeDtypeStruct((_NW, _NCH, _CH, _EMB), jnp.float32),
            ),
            mesh=mesh,
            scratch_types=[
                pltpu.VMEM((_NCH, _CH), jnp.int32),
                pltpu.VMEM((_NCH, _CH), jnp.int32),
                pltpu.VMEM((_NCH, _CH, _EMB), jnp.float32),
                pltpu.VMEM((_NCH, _CH, _EMB), jnp.float32),
                pltpu.SemaphoreType.DMA,
            ],
            compiler_params=pltpu.CompilerParams(use_tc_tiling_on_sc=False),
        ))
    return _gather_cache[0](*args)


def _mlp_body(s, t, n, w1s, w1t, w1n, b1, w2, b2, o):
    h = (jnp.dot(s[...], w1s[...], preferred_element_type=jnp.float32)
         + jnp.dot(t[...], w1t[...], preferred_element_type=jnp.float32)
         + jnp.dot(n[...], w1n[...], preferred_element_type=jnp.float32)
         + b1[...])
    h = jnp.maximum(h, 0.0)
    o[...] = jnp.dot(h, w2[...], preferred_element_type=jnp.float32) + b2[...]


_BLK = 2048


def _mlp(s, t, n, w1s, w1t, w1n, b1, w2, b2):
    grid = (_B // _BLK,)
    full = lambda i: (0, 0)
    return pl.pallas_call(
        _mlp_body,
        grid=grid,
        in_specs=[
            pl.BlockSpec((_BLK, _EMB), lambda i: (i, 0)),
            pl.BlockSpec((_BLK, _EMB), lambda i: (i, 0)),
            pl.BlockSpec((_BLK, _NUMF), lambda i: (i, 0)),
            pl.BlockSpec((_EMB, _HID), full),
            pl.BlockSpec((_EMB, _HID), full),
            pl.BlockSpec((_NUMF, _HID), full),
            pl.BlockSpec((1, _HID), full),
            pl.BlockSpec((_HID, _NCLS), full),
            pl.BlockSpec((1, _NCLS), full),
        ],
        out_specs=pl.BlockSpec((_BLK, _NCLS), lambda i: (i, 0)),
        out_shape=jax.ShapeDtypeStruct((_B, _NCLS), jnp.float32),
    )(s, t, n, w1s, w1t, w1n, b1, w2, b2)


def kernel(cat_feats, num_feats, src_emb, tgt_emb, W1, b1, W2, b2):
    src_id = cat_feats[:, 0].reshape(_NW, _NCH, _CH)
    tgt_id = cat_feats[:, 1].reshape(_NW, _NCH, _CH)
    srows, trows = _gather(src_id, tgt_id, src_emb, tgt_emb)
    s = srows.reshape(_B, _EMB)
    t = trows.reshape(_B, _EMB)
    w1s = W1[:, :_EMB].T
    w1t = W1[:, _EMB:2 * _EMB].T
    w1n = W1[:, 2 * _EMB:].T
    return _mlp(s, t, num_feats, w1s, w1t, w1n,
                b1.reshape(1, _HID), W2.T, b2.reshape(1, _NCLS))
